# baseline (speedup 1.0000x reference)
import functools

import jax
import jax.numpy as jnp
from jax import lax
from jax.experimental import pallas as pl
from jax.experimental.pallas import tpu as pltpu
from jax.experimental.pallas import tpu_sc as plsc

FILE_VOCAB = 100000
TEST_VOCAB = 100000
EMB = 64
BATCH = 16384

NC = 2
NS = 16
NW = NC * NS
B_PER_W = BATCH // NW
CHUNK = 16
N_CHUNKS = B_PER_W // CHUNK


def _rsqrt(x):
    i = lax.bitcast_convert_type(x, jnp.int32)
    i = jnp.int32(0x5F3759DF) - lax.shift_right_arithmetic(i, jnp.int32(1))
    y = lax.bitcast_convert_type(i, jnp.float32)
    half_x = x * 0.5
    for _ in range(3):
        y = y * (1.5 - half_x * y * y)
    return y


def _body(fidx_hbm, tidx_hbm, ftab_hbm, ttab_hbm, wb_hbm, out_hbm,
          fidx_v, tidx_v, frows_v, trows_v, wb_v, out_v, sem_f, sem_t):
    wid = lax.axis_index("s") * NC + lax.axis_index("c")
    base = wid * B_PER_W

    pltpu.sync_copy(fidx_hbm.at[pl.ds(base, B_PER_W)], fidx_v)
    pltpu.sync_copy(tidx_hbm.at[pl.ds(base, B_PER_W)], tidx_v)
    pltpu.sync_copy(wb_hbm, wb_v)

    cp_f = pltpu.async_copy(ftab_hbm.at[fidx_v], frows_v, sem_f)
    cp_t = pltpu.async_copy(ttab_hbm.at[tidx_v], trows_v, sem_t)
    cp_f.wait()
    cp_t.wait()

    lanes = lax.iota(jnp.int32, 16)

    def chunk_body(c, _):
        zero = jnp.zeros((16,), jnp.float32)
        acc_ab, acc_aa, acc_bb = zero, zero, zero
        for j in range(CHUNK):
            row = c * CHUNK + j
            ab = zero
            aa = zero
            bb = zero
            for k in range(EMB // 16):
                fa = frows_v[row, pl.ds(k * 16, 16)]
                ta = trows_v[row, pl.ds(k * 16, 16)]
                ab = ab + fa * ta
                aa = aa + fa * fa
                bb = bb + ta * ta
            lane_j = lanes == j
            acc_ab = jnp.where(lane_j, jnp.sum(ab), acc_ab)
            acc_aa = jnp.where(lane_j, jnp.sum(aa), acc_aa)
            acc_bb = jnp.where(lane_j, jnp.sum(bb), acc_bb)

        r = _rsqrt(jnp.maximum(acc_aa, 1e-24) * jnp.maximum(acc_bb, 1e-24))
        merged = acc_ab * r
        w = wb_v[pl.ds(0, 16)]
        bias = wb_v[pl.ds(16, 16)]
        z = merged * w + bias
        out = 1.0 / (1.0 + jnp.exp(-z))
        out_v[pl.ds(c * CHUNK, CHUNK)] = out
        return ()

    lax.fori_loop(0, N_CHUNKS, chunk_body, (), unroll=False)

    pltpu.sync_copy(out_v, out_hbm.at[pl.ds(base, B_PER_W)])


@functools.partial(jax.jit, static_argnames=())
def _run(fidx, tidx, ftab, ttab, wb):
    mesh = plsc.VectorSubcoreMesh(
        core_axis_name="c", subcore_axis_name="s",
        num_cores=NC, num_subcores=NS)
    return pl.kernel(
        _body,
        out_type=jax.ShapeDtypeStruct((BATCH,), jnp.float32),
        mesh=mesh,
        compiler_params=pltpu.CompilerParams(
            needs_layout_passes=False, use_tc_tiling_on_sc=False),
        scratch_types=[
            pltpu.VMEM((B_PER_W,), jnp.int32),
            pltpu.VMEM((B_PER_W,), jnp.int32),
            pltpu.VMEM((B_PER_W, EMB), jnp.float32),
            pltpu.VMEM((B_PER_W, EMB), jnp.float32),
            pltpu.VMEM((32,), jnp.float32),
            pltpu.VMEM((B_PER_W,), jnp.float32),
            pltpu.SemaphoreType.DMA,
            pltpu.SemaphoreType.DMA,
        ],
    )(fidx, tidx, ftab, ttab, wb)


def kernel(file, test, file_emb, test_emb, W, b):
    fidx = file.reshape(BATCH).astype(jnp.int32)
    tidx = test.reshape(BATCH).astype(jnp.int32)
    wb = jnp.concatenate([
        jnp.broadcast_to(W.reshape(1), (16,)),
        jnp.broadcast_to(b.reshape(1), (16,)),
    ]).astype(jnp.float32)
    out = _run(fidx, tidx, file_emb, test_emb, wb)
    return out.reshape(BATCH, 1)
